# Initial kernel scaffold; baseline (speedup 1.0000x reference)
import jax
import jax.numpy as jnp
from jax.experimental import pallas as pl


def kernel(x, weight, bias):
    raise NotImplementedError("write your pallas kernel here")


# speedup vs baseline: 1.3806x; 1.3806x over previous
import functools

import jax
import jax.numpy as jnp
from jax.experimental import pallas as pl
from jax.experimental.pallas import tpu as pltpu


def _phase_kernel(x_ref, w_ref, b_ref, o_ref, pad_ref, *, H, W, C, th):
    W2 = 2 * W
    r0 = pl.program_id(1) * th

    pad_ref[...] = jnp.zeros_like(pad_ref)
    pad_ref[1:1 + th, 1:1 + W, :] = x_ref[pl.ds(r0, th)]

    @pl.when(r0 > 0)
    def _top():
        pad_ref[0, 1:1 + W, :] = x_ref[r0 - 1]

    @pl.when(r0 + th < H)
    def _bot():
        pad_ref[th + 1, 1:1 + W, :] = x_ref[r0 + th]

    pix = th * W
    patches = [[pad_ref[a:a + th, b:b + W, :].reshape(pix, C)
                for b in range(3)] for a in range(3)]
    bias0 = b_ref[...]
    out_rows = []
    for di in range(2):
        cols = []
        for dj in range(2):
            acc = jnp.zeros((pix, C), jnp.float32) + bias0
            for ty in range(2):
                for tx in range(2):
                    k = ((di * 2 + dj) * 2 + ty) * 2 + tx
                    acc = acc + jnp.dot(patches[di + ty][dj + tx], w_ref[k],
                                        preferred_element_type=jnp.float32)
            cols.append(acc.reshape(th, W, C))
        out_rows.append(jnp.stack(cols, axis=2).reshape(th, W2, C))
    y = jnp.stack(out_rows, axis=1).reshape(2 * th * W2, C)
    o_ref[...] = y.astype(o_ref.dtype)


def kernel(x, weight, bias):
    N, C, H, W = x.shape
    H2, W2 = 2 * H, 2 * W
    th = next(t for t in (16, 8, 4, 2, 1) if H % t == 0)
    n_bands = H // th

    x_nhwc = jnp.transpose(x, (0, 2, 3, 1)).reshape(N * H, W, C)
    w33 = jnp.transpose(weight, (2, 3, 1, 0))
    R = jnp.array([[[1, 0, 0], [0, 1, 1]],
                   [[1, 1, 0], [0, 0, 1]]], x.dtype)
    wc = jnp.einsum('ayh,bxw,hwio->abyxio', R, R, w33).reshape(16, C, C)
    wc = wc.astype(x.dtype)
    b2 = bias.reshape(1, C).astype(jnp.float32)

    out = pl.pallas_call(
        functools.partial(_phase_kernel, H=H, W=W, C=C, th=th),
        out_shape=jax.ShapeDtypeStruct((N * H2 * W2, C), x.dtype),
        grid_spec=pltpu.PrefetchScalarGridSpec(
            num_scalar_prefetch=0,
            grid=(N, n_bands),
            in_specs=[
                pl.BlockSpec((H, W, C), lambda n, r: (n, 0, 0)),
                pl.BlockSpec((16, C, C), lambda n, r: (0, 0, 0)),
                pl.BlockSpec((1, C), lambda n, r: (0, 0)),
            ],
            out_specs=pl.BlockSpec((2 * th * W2, C),
                                   lambda n, r: (n * n_bands + r, 0)),
            scratch_shapes=[pltpu.VMEM((th + 2, W + 2, C), x.dtype)],
        ),
        compiler_params=pltpu.CompilerParams(
            dimension_semantics=("parallel", "arbitrary"),
            vmem_limit_bytes=64 << 20),
    )(x_nhwc, wc, b2)
    return jnp.transpose(out.reshape(N, H2, W2, C), (0, 3, 1, 2))


# speedup vs baseline: 1.6135x; 1.1687x over previous
import functools

import jax
import jax.numpy as jnp
from jax import lax
from jax.experimental import pallas as pl
from jax.experimental.pallas import tpu as pltpu


def _phase_kernel(x_ref, w_ref, b_ref, o_ref, xcol_ref, *, H, W, C, th):
    W2 = 2 * W
    pix = th * W
    n_bands = H // th
    PW = (H + 2) * W

    xt = jnp.transpose(x_ref[0].astype(jnp.bfloat16))
    zrow = jnp.zeros((W, C), jnp.bfloat16)
    xtp = jnp.concatenate([zrow, xt, zrow], axis=0)
    iota = lax.broadcasted_iota(jnp.int32, (PW, 1), 0)
    z1 = jnp.zeros((1, C), jnp.bfloat16)
    xl = jnp.where(iota % W == 0, 0,
                   jnp.concatenate([z1, xtp[:-1]], axis=0))
    xr = jnp.where(iota % W == W - 1, 0,
                   jnp.concatenate([xtp[1:], z1], axis=0))
    xcol_ref[...] = jnp.concatenate([xl, xtp, xr], axis=1)

    bias0 = b_ref[...]
    for r in range(n_bands):
        base = (1 + r * th) * W
        ys = []
        for di in range(2):
            acc = jnp.zeros((pix, 2 * C), jnp.float32) + bias0
            for ty in range(2):
                s = base + (di + ty - 1) * W
                acc = acc + jnp.dot(xcol_ref[s:s + pix], w_ref[di * 2 + ty],
                                    preferred_element_type=jnp.float32)
            ys.append(acc.reshape(2 * pix, C).reshape(th, W2, C))
        y = jnp.stack(ys, axis=1).reshape(2 * th * W2, C)
        o_ref[0, :, r * 2 * th * W2:(r + 1) * 2 * th * W2] = (
            jnp.transpose(y.astype(o_ref.dtype)))


def kernel(x, weight, bias):
    N, C, H, W = x.shape
    H2, W2 = 2 * H, 2 * W
    th = next(t for t in (16, 8, 4, 2, 1) if H % t == 0)

    w33 = jnp.transpose(weight, (2, 3, 1, 0))
    R = jnp.array([[[1, 0, 0], [0, 1, 1]],
                   [[1, 1, 0], [0, 0, 1]]], jnp.float32)
    wc = jnp.einsum('ayh,bxw,hwio->abyxio', R, R, w33)
    w3 = jnp.zeros((2, 2, 3 * C, 2 * C), jnp.float32)
    for di in range(2):
        for ty in range(2):
            for dj in range(2):
                for tx in range(2):
                    s = dj + tx
                    w3 = w3.at[di, ty, s * C:(s + 1) * C,
                               dj * C:(dj + 1) * C].set(wc[di, dj, ty, tx])
    w3 = w3.reshape(4, 3 * C, 2 * C).astype(jnp.bfloat16)
    b2 = jnp.tile(bias.reshape(1, C), (1, 2)).astype(jnp.float32)

    out = pl.pallas_call(
        functools.partial(_phase_kernel, H=H, W=W, C=C, th=th),
        out_shape=jax.ShapeDtypeStruct((N, C, H2 * W2), x.dtype),
        grid_spec=pltpu.PrefetchScalarGridSpec(
            num_scalar_prefetch=0,
            grid=(N,),
            in_specs=[
                pl.BlockSpec((1, C, H * W), lambda n: (n, 0, 0)),
                pl.BlockSpec((4, 3 * C, 2 * C), lambda n: (0, 0, 0)),
                pl.BlockSpec((1, 2 * C), lambda n: (0, 0)),
            ],
            out_specs=pl.BlockSpec((1, C, H2 * W2), lambda n: (n, 0, 0)),
            scratch_shapes=[
                pltpu.VMEM(((H + 2) * W, 3 * C), jnp.bfloat16),
            ],
        ),
        compiler_params=pltpu.CompilerParams(
            dimension_semantics=("parallel",),
            vmem_limit_bytes=100 << 20),
    )(x.reshape(N, C, H * W), w3, b2)
    return out.reshape(N, C, H2, W2)
